# baseline (speedup 1.0000x reference)
import jax
import jax.numpy as jnp
from jax import lax
from jax.experimental import pallas as pl
from jax.experimental.pallas import tpu as pltpu
from jax.experimental.pallas import tpu_sc as plsc

K = 256
G = 2048
X0 = -8.0
X1 = 8.0
H = (X1 - X0) / (G - 1)
INV_H = 1.0 / H

NC, NS, L = 2, 16, 16
NW = NC * NS
GT = G // NS
GV = GT // L


def _make_sc_kernel(n):
    per_w = n // NW
    vecs = per_w // L
    mesh = plsc.VectorSubcoreMesh(core_axis_name="c", subcore_axis_name="s",
                                  num_cores=NC, num_subcores=NS)

    def _body(x_hbm, lev_hbm, beta_hbm, out_hbm,
              lev_v, beta_v, slice_v, tab_v, x_v, y_v, shared_tab, xsem):
        c = lax.axis_index("c")
        s = lax.axis_index("s")
        wid = c * NS + s
        base = wid * per_w

        xcopy = pltpu.make_async_copy(x_hbm.at[pl.ds(base, per_w)], x_v, xsem)
        xcopy.start()
        pltpu.sync_copy(lev_hbm, lev_v.at[pl.ds(0, K)])
        pltpu.sync_copy(beta_hbm, beta_v)

        beta_vec = beta_v[pl.ds(0, L)]
        beta = beta_vec[0]
        lv0 = lev_v[pl.ds(0, L)]
        l0 = lv0[0]
        lvlast = lev_v[pl.ds(K - L, L)]
        lev_v[pl.ds(K, L)] = jnp.full((L,), lvlast[L - 1], jnp.float32)

        gbase = s * GT
        bxgs = tuple(
            (X0 * beta) + (H * beta)
            * (lax.iota(jnp.int32, L) + (gbase + j * L)).astype(jnp.float32)
            for j in range(GV)
        )

        def term(c, accs):
            lv = lev_v[pl.ds(c * L, L)]
            lv1 = lev_v[pl.ds(c * L + 1, L)]
            kidx = lax.iota(jnp.int32, L) + c * L
            wv = jnp.where(kidx < K - 1, lv1 - lv, 0.0)
            bbv = (0.5 * beta) * (lv1 + lv)
            new = list(accs)
            for lane in range(L):
                w = wv[lane]
                bb = bbv[lane]
                new = [acc + w / (1.0 + jnp.exp(bb - bxg))
                       for acc, bxg in zip(new, bxgs)]
            return tuple(new)

        init = tuple(jnp.full((L,), l0, jnp.float32) for _ in range(GV))
        accs = lax.fori_loop(0, L, term, init)
        for j in range(GV):
            slice_v[pl.ds(j * L, L)] = accs[j]

        pltpu.sync_copy(slice_v, shared_tab.at[pl.ds(gbase, GT)])
        plsc.subcore_barrier()
        pltpu.sync_copy(shared_tab, tab_v)
        xcopy.wait()

        def body(i, carry):
            xv = x_v[pl.ds(i * L, L)]
            t = (jnp.clip(xv, X0, X1) - X0) * INV_H
            idx = jnp.minimum(t.astype(jnp.int32), G - 2)
            fr = t - idx.astype(jnp.float32)
            y0 = plsc.load_gather(tab_v, [idx])
            y1 = plsc.load_gather(tab_v, [idx + 1])
            y_v[pl.ds(i * L, L)] = y0 + fr * (y1 - y0)
            return carry

        lax.fori_loop(0, vecs, body, 0)
        pltpu.sync_copy(y_v, out_hbm.at[pl.ds(base, per_w)])

    return pl.kernel(
        _body,
        out_type=jax.ShapeDtypeStruct((n,), jnp.float32),
        mesh=mesh,
        scratch_types=[
            pltpu.VMEM((K + L,), jnp.float32),
            pltpu.VMEM((L,), jnp.float32),
            pltpu.VMEM((GT,), jnp.float32),
            pltpu.VMEM((G,), jnp.float32),
            pltpu.VMEM((per_w,), jnp.float32),
            pltpu.VMEM((per_w,), jnp.float32),
            pltpu.VMEM_SHARED((G,), jnp.float32),
            pltpu.SemaphoreType.DMA,
        ],
        compiler_params=pltpu.CompilerParams(needs_layout_passes=False),
    )


def kernel(x, levels, beta):
    beta16 = jnp.full((L,), jnp.reshape(beta, ()), dtype=jnp.float32)
    xf = x.reshape(-1)
    y = _make_sc_kernel(xf.size)(xf, levels, beta16)
    return y.reshape(x.shape)


# speedup vs baseline: 1.6521x; 1.6521x over previous
import jax
import jax.numpy as jnp
from jax import lax
from jax.experimental import pallas as pl
from jax.experimental.pallas import tpu as pltpu
from jax.experimental.pallas import tpu_sc as plsc

K = 256
G = 2048
X0 = -8.0
X1 = 8.0
H = (X1 - X0) / (G - 1)
INV_H = 1.0 / H
GR = G // 128
KU = 3

NC, NS, L = 2, 16, 16
NW = NC * NS


def _tab_body(lev_ref, beta_ref, tab_ref):
    beta = beta_ref[0]
    l0 = lev_ref[0]
    gidx = (lax.broadcasted_iota(jnp.int32, (GR, 128), 0) * 128
            + lax.broadcasted_iota(jnp.int32, (GR, 128), 1))
    xg = X0 + H * gidx.astype(jnp.float32)

    def body(j, acc):
        for u in range(KU):
            k = j * KU + u
            lk = lev_ref[k]
            lk1 = lev_ref[k + 1]
            w = lk1 - lk
            b = 0.5 * (lk1 + lk)
            acc = acc + w * jax.nn.sigmoid(beta * (xg - b))
        return acc

    init = jnp.full((GR, 128), l0, jnp.float32)
    tab_ref[...] = lax.fori_loop(0, (K - 1) // KU, body, init)


def _make_sc_lookup(b_dim, r_dim, c_dim):
    rows_w = (b_dim * r_dim) // NW
    cv = c_dim // L
    w_per_b = r_dim // rows_w
    mesh = plsc.VectorSubcoreMesh(core_axis_name="c", subcore_axis_name="s",
                                  num_cores=NC, num_subcores=NS)

    def _sc_body(x_hbm, tab_hbm, out_hbm, tab_v, x_v, y_v, xsem):
        wid = lax.axis_index("c") * NS + lax.axis_index("s")
        b = wid // w_per_b
        r0 = (wid % w_per_b) * rows_w

        xcopy = pltpu.make_async_copy(
            x_hbm.at[b, pl.ds(r0, rows_w), :], x_v, xsem)
        xcopy.start()
        pltpu.sync_copy(tab_hbm, tab_v)
        xcopy.wait()

        @plsc.parallel_loop(0, rows_w, unroll=2)
        def row(r):
            for c in range(cv):
                xv = x_v[r, pl.ds(c * L, L)]
                t = (jnp.clip(xv, X0, X1) - X0) * INV_H
                idx = jnp.minimum(t.astype(jnp.int32), G - 2)
                fr = t - idx.astype(jnp.float32)
                y0 = plsc.load_gather(tab_v, [idx])
                y1 = plsc.load_gather(tab_v, [idx + 1])
                y_v[r, pl.ds(c * L, L)] = y0 + fr * (y1 - y0)

        pltpu.sync_copy(y_v, out_hbm.at[b, pl.ds(r0, rows_w), :])

    return pl.kernel(
        _sc_body,
        out_type=jax.ShapeDtypeStruct((b_dim, r_dim, c_dim), jnp.float32),
        mesh=mesh,
        scratch_types=[
            pltpu.VMEM((G,), jnp.float32),
            pltpu.VMEM((rows_w, c_dim), jnp.float32),
            pltpu.VMEM((rows_w, c_dim), jnp.float32),
            pltpu.SemaphoreType.DMA,
        ],
        compiler_params=pltpu.CompilerParams(needs_layout_passes=False),
    )


def kernel(x, levels, beta):
    beta_arr = jnp.reshape(beta, (1,)).astype(jnp.float32)
    tab2d = pl.pallas_call(
        _tab_body,
        out_shape=jax.ShapeDtypeStruct((GR, 128), jnp.float32),
        in_specs=[pl.BlockSpec(memory_space=pltpu.SMEM),
                  pl.BlockSpec(memory_space=pltpu.SMEM)],
        out_specs=pl.BlockSpec(memory_space=pltpu.VMEM),
    )(levels, beta_arr)
    tab = tab2d.reshape(G)

    b_dim, r_dim, c_dim = x.shape
    return _make_sc_lookup(b_dim, r_dim, c_dim)(x, tab)
